# baseline (speedup 1.0000x reference)
import jax
import jax.numpy as jnp
from jax.experimental import pallas as pl
from jax.experimental.pallas import tpu as pltpu

_N = 4096
_D = 2048
_E = 8
_H = 256
_O = 2048
_LANES = 128
_BN = 512


def _dot_t(a, b):
    return jax.lax.dot_general(
        a, b, (((1,), (1,)), ((), ())), preferred_element_type=jnp.float32
    )


def _gate_weights(x, gwt, gb):
    logits = _dot_t(x, gwt)
    lane = jax.lax.broadcasted_iota(jnp.int32, logits.shape, 1)
    valid = lane < _E
    l = jnp.where(valid, logits + gb, -jnp.inf)
    m = jnp.max(l, axis=1, keepdims=True)
    p = jnp.where(valid, jnp.exp(l - m), 0.0)
    probs = p / jnp.sum(p, axis=1, keepdims=True)
    m1 = jnp.max(probs, axis=1, keepdims=True)
    a1 = jnp.min(jnp.where((probs == m1) & valid, lane, _LANES), axis=1, keepdims=True)
    probs2 = jnp.where(lane == a1, -1.0, probs)
    m2 = jnp.max(probs2, axis=1, keepdims=True)
    a2 = jnp.min(jnp.where((probs2 == m2) & valid, lane, _LANES), axis=1, keepdims=True)
    denom = m1 + m2
    w = jnp.where(lane == a1, m1 / denom, 0.0) + jnp.where(lane == a2, m2 / denom, 0.0)
    return w


def _moe_body(x_ref, gwt_ref, gb_ref, w1f_ref, b1f_ref, w2t_ref, b2p_ref, out_ref):
    x = x_ref[...]
    w = _gate_weights(x, gwt_ref[...], gb_ref[...])
    h = jax.nn.relu(_dot_t(x, w1f_ref[...]) + b1f_ref[...])
    hw = jnp.concatenate(
        [h[:, e * _H : (e + 1) * _H] * w[:, e : e + 1] for e in range(_E)], axis=1
    )
    acc = jnp.dot(hw, w2t_ref[...], preferred_element_type=jnp.float32)
    acc = acc + jnp.dot(w, b2p_ref[...], preferred_element_type=jnp.float32)
    out_ref[...] = acc.astype(jnp.bfloat16)


def _heads_body(eo_ref, cw_ref, cb_ref, vw_ref, vb_ref, cls_ref, vec_ref):
    eo = eo_ref[...].astype(jnp.float32)
    cls_ref[...] = _dot_t(eo, cw_ref[...]) + cb_ref[...]
    vec_ref[...] = _dot_t(eo, vw_ref[...]) + vb_ref[...]


def kernel(x, gate_W, gate_b, W1, b1, W2, b2, cls_W, cls_b, vec_W, vec_b):
    x = x.astype(jnp.float32)
    gwt = jnp.pad(gate_W, ((0, _LANES - _E), (0, 0)))
    gb = jnp.pad(gate_b, (0, _LANES - _E))[None, :]
    w1f = W1.reshape(_E * _H, _D)
    b1f = b1.reshape(1, _E * _H)
    w2t = W2.transpose(0, 2, 1).reshape(_E * _H, _O)
    b2p = jnp.pad(b2, ((0, _LANES - _E), (0, 0)))

    grid = _N // _BN
    whole = lambda shape: pl.BlockSpec(shape, lambda i: (0,) * len(shape))

    eo = pl.pallas_call(
        _moe_body,
        grid=(grid,),
        in_specs=[
            pl.BlockSpec((_BN, _D), lambda i: (i, 0)),
            whole((_LANES, _D)),
            whole((1, _LANES)),
            whole((_E * _H, _D)),
            whole((1, _E * _H)),
            whole((_E * _H, _O)),
            whole((_LANES, _O)),
        ],
        out_specs=pl.BlockSpec((_BN, _O), lambda i: (i, 0)),
        out_shape=jax.ShapeDtypeStruct((_N, _O), jnp.bfloat16),
        compiler_params=pltpu.CompilerParams(
            vmem_limit_bytes=120 * 1024 * 1024,
        ),
    )(x, gwt, gb, w1f, b1f, w2t, b2p)

    cls_out, vec_out = pl.pallas_call(
        _heads_body,
        grid=(grid,),
        in_specs=[
            pl.BlockSpec((_BN, _O), lambda i: (i, 0)),
            whole((_O, _O)),
            whole((1, _O)),
            whole((_O, _O)),
            whole((1, _O)),
        ],
        out_specs=[
            pl.BlockSpec((_BN, _O), lambda i: (i, 0)),
            pl.BlockSpec((_BN, _O), lambda i: (i, 0)),
        ],
        out_shape=[
            jax.ShapeDtypeStruct((_N, _O), jnp.float32),
            jax.ShapeDtypeStruct((_N, _O), jnp.float32),
        ],
        compiler_params=pltpu.CompilerParams(
            vmem_limit_bytes=120 * 1024 * 1024,
        ),
    )(eo, cls_W, cls_b[None, :], vec_W, vec_b[None, :])

    return (cls_out, vec_out)


# speedup vs baseline: 1.2505x; 1.2505x over previous
import jax
import jax.numpy as jnp
from jax.experimental import pallas as pl
from jax.experimental.pallas import tpu as pltpu

_N = 4096
_D = 2048
_E = 8
_H = 256
_O = 2048
_LANES = 128
_BN = 512


def _dot_t(a, b):
    return jax.lax.dot_general(
        a, b, (((1,), (1,)), ((), ())), preferred_element_type=jnp.float32
    )


def _gate_weights(x, gwt, gb):
    logits = _dot_t(x, gwt)
    lane = jax.lax.broadcasted_iota(jnp.int32, logits.shape, 1)
    valid = lane < _E
    l = jnp.where(valid, logits + gb, -jnp.inf)
    m = jnp.max(l, axis=1, keepdims=True)
    p = jnp.where(valid, jnp.exp(l - m), 0.0)
    probs = p / jnp.sum(p, axis=1, keepdims=True)
    m1 = jnp.max(probs, axis=1, keepdims=True)
    a1 = jnp.min(jnp.where((probs == m1) & valid, lane, _LANES), axis=1, keepdims=True)
    probs2 = jnp.where(lane == a1, -1.0, probs)
    m2 = jnp.max(probs2, axis=1, keepdims=True)
    a2 = jnp.min(jnp.where((probs2 == m2) & valid, lane, _LANES), axis=1, keepdims=True)
    denom = m1 + m2
    w = jnp.where(lane == a1, m1 / denom, 0.0) + jnp.where(lane == a2, m2 / denom, 0.0)
    return w


def _moe_body(x_ref, gwt_ref, gb_ref, w1f_ref, b1f_ref, w2t_ref, b2p_ref, exp_ref, out_ref):
    x = x_ref[...]
    w = _gate_weights(x, gwt_ref[...], gb_ref[...])
    h = jax.nn.relu(_dot_t(x, w1f_ref[...]) + b1f_ref[...])
    wexp = jnp.dot(w, exp_ref[...], preferred_element_type=jnp.float32)
    acc = jnp.dot(h * wexp, w2t_ref[...], preferred_element_type=jnp.float32)
    acc = acc + jnp.dot(w, b2p_ref[...], preferred_element_type=jnp.float32)
    out_ref[...] = acc


def _heads_body(eo_ref, cw_ref, cb_ref, vw_ref, vb_ref, cls_ref, vec_ref):
    eo = eo_ref[...]
    cls_ref[...] = _dot_t(eo, cw_ref[...]) + cb_ref[...]
    vec_ref[...] = _dot_t(eo, vw_ref[...]) + vb_ref[...]


def kernel(x, gate_W, gate_b, W1, b1, W2, b2, cls_W, cls_b, vec_W, vec_b):
    x = x.astype(jnp.float32)
    gwt = jnp.pad(gate_W, ((0, _LANES - _E), (0, 0)))
    gb = jnp.pad(gate_b, (0, _LANES - _E))[None, :]
    w1f = W1.reshape(_E * _H, _D)
    b1f = b1.reshape(1, _E * _H)
    w2t = W2.transpose(0, 2, 1).reshape(_E * _H, _O)
    b2p = jnp.pad(b2, ((0, _LANES - _E), (0, 0)))
    expand = (
        jax.lax.broadcasted_iota(jnp.int32, (_LANES, _E * _H), 1) // _H
        == jax.lax.broadcasted_iota(jnp.int32, (_LANES, _E * _H), 0)
    ).astype(jnp.float32)

    grid = _N // _BN
    whole = lambda shape: pl.BlockSpec(shape, lambda i: (0,) * len(shape))

    eo = pl.pallas_call(
        _moe_body,
        grid=(grid,),
        in_specs=[
            pl.BlockSpec((_BN, _D), lambda i: (i, 0)),
            whole((_LANES, _D)),
            whole((1, _LANES)),
            whole((_E * _H, _D)),
            whole((1, _E * _H)),
            whole((_E * _H, _O)),
            whole((_LANES, _O)),
            whole((_LANES, _E * _H)),
        ],
        out_specs=pl.BlockSpec((_BN, _O), lambda i: (i, 0)),
        out_shape=jax.ShapeDtypeStruct((_N, _O), jnp.float32),
        compiler_params=pltpu.CompilerParams(
            vmem_limit_bytes=120 * 1024 * 1024,
        ),
    )(x, gwt, gb, w1f, b1f, w2t, b2p, expand)

    cls_out, vec_out = pl.pallas_call(
        _heads_body,
        grid=(grid,),
        in_specs=[
            pl.BlockSpec((_BN, _O), lambda i: (i, 0)),
            whole((_O, _O)),
            whole((1, _O)),
            whole((_O, _O)),
            whole((1, _O)),
        ],
        out_specs=[
            pl.BlockSpec((_BN, _O), lambda i: (i, 0)),
            pl.BlockSpec((_BN, _O), lambda i: (i, 0)),
        ],
        out_shape=[
            jax.ShapeDtypeStruct((_N, _O), jnp.float32),
            jax.ShapeDtypeStruct((_N, _O), jnp.float32),
        ],
        compiler_params=pltpu.CompilerParams(
            vmem_limit_bytes=120 * 1024 * 1024,
        ),
    )(eo, cls_W, cls_b[None, :], vec_W, vec_b[None, :])

    return (cls_out, vec_out)
